# baseline (speedup 1.0000x reference)
import functools

import jax
import jax.numpy as jnp
from jax import lax
from jax.experimental import pallas as pl
from jax.experimental.pallas import tpu as pltpu
from jax.experimental.pallas import tpu_sc as plsc

_MAX_MASK_PCT = 0.15
_NUM_MASKS = 2
_B, _P, _D = 16, 2048, 1024
_NW = 32
_RPW = _B * _P // _NW
_CHUNK = 32
_NCHUNK = _RPW // _CHUNK


def _segment_bounds(X_len):
    rk = jax.random.key(42)
    ka, kb = jax.random.split(rk)
    valid = X_len
    mml = jnp.floor(_MAX_MASK_PCT * valid.astype(jnp.float32)).astype(jnp.int32)
    vrep = jnp.repeat(valid, _NUM_MASKS)
    mrep = jnp.repeat(mml, _NUM_MASKS)
    n = _B * _NUM_MASKS
    t = jnp.floor(jax.random.uniform(ka, (n,)) * (mrep + 1).astype(jnp.float32)).astype(jnp.int32)
    max_start = jnp.clip(vrep - t + 1, 1, None)
    t0 = jnp.floor(jax.random.uniform(kb, (n,)) * max_start.astype(jnp.float32)).astype(jnp.int32)
    t1 = t0 + t
    segs = jnp.stack(
        [t0.reshape(_B, _NUM_MASKS), t1.reshape(_B, _NUM_MASKS)], axis=-1
    ).reshape(_B, 4)
    return jnp.repeat(jnp.pad(segs, ((0, 0), (0, 12))), _NW // _B, axis=0)


_mesh = plsc.VectorSubcoreMesh(core_axis_name="c", subcore_axis_name="s")


@functools.partial(
    pl.kernel,
    mesh=_mesh,
    out_type=[
        jax.ShapeDtypeStruct((_B * _P, _D), jnp.float32),
        jax.ShapeDtypeStruct((_B * _P,), jnp.int32),
    ],
    scratch_types=[
        pltpu.VMEM((16,), jnp.float32),
        pltpu.VMEM((1, _D), jnp.float32),
        pltpu.VMEM((_RPW,), jnp.int32),
        pltpu.VMEM((16,), jnp.int32),
        pltpu.SemaphoreType.DMA,
    ],
)
def _sc_masked_copy(x_hbm, segs_hbm, mval_hbm, out_hbm, mask_hbm,
                    mval_v, mvrow, maskbuf, segs_v, sem):
    wid = lax.axis_index("s") * 2 + lax.axis_index("c")
    base = wid * _RPW
    p0 = (wid % 2) * _RPW

    pltpu.sync_copy(segs_hbm.at[wid], segs_v)
    pltpu.sync_copy(mval_hbm, mval_v)
    sv = segs_v[:]
    s0 = sv[0]
    e0 = sv[1]
    s1 = sv[2]
    e1 = sv[3]

    mv = mval_v[:]
    for i in range(_D // 16):
        mvrow[0, pl.ds(16 * i, 16)] = mv

    for i in range(_NCHUNK):
        lo = p0 + i * _CHUNK
        hi = lo + _CHUNK
        r0 = base + i * _CHUNK
        inside = ((lo >= s0) & (hi <= e0)) | ((lo >= s1) & (hi <= e1))
        clear0 = (hi <= s0) | (lo >= e0) | (e0 <= s0)
        clear1 = (hi <= s1) | (lo >= e1) | (e1 <= s1)
        untouched = clear0 & clear1

        @pl.when(untouched)
        def _():
            pltpu.async_copy(
                x_hbm.at[pl.ds(r0, _CHUNK)], out_hbm.at[pl.ds(r0, _CHUNK)], sem
            )

        @pl.when(jnp.logical_not(untouched))
        def _():
            def row(j, c):
                p = lo + j
                masked = ((p >= s0) & (p < e0)) | ((p >= s1) & (p < e1))

                @pl.when(masked)
                def _():
                    pltpu.async_copy(mvrow, out_hbm.at[pl.ds(r0 + j, 1)], sem)

                @pl.when(jnp.logical_not(masked))
                def _():
                    pltpu.async_copy(
                        x_hbm.at[pl.ds(r0 + j, 1)],
                        out_hbm.at[pl.ds(r0 + j, 1)],
                        sem,
                    )

                return c

            lax.fori_loop(0, _CHUNK, row, 0)

    one16 = jnp.full((16,), 1, jnp.int32)
    zero16 = jnp.zeros((16,), jnp.int32)

    def mrow(i, c):
        p = p0 + i * 16 + lax.iota(jnp.int32, 16)
        m = ((p >= s0) & (p < e0)) | ((p >= s1) & (p < e1))
        maskbuf[pl.ds(i * 16, 16)] = jnp.where(m, one16, zero16)
        return c

    lax.fori_loop(0, _RPW // 16, mrow, 0)
    pltpu.sync_copy(maskbuf, mask_hbm.at[pl.ds(base, _RPW)])

    pltpu.make_async_copy(
        x_hbm.at[pl.ds(base, _RPW)], out_hbm.at[pl.ds(base, _RPW)], sem
    ).wait()


def kernel(X, X_len, mask_value):
    segs = _segment_bounds(X_len)
    mval16 = jnp.full((16,), mask_value[0], jnp.float32)
    xf = X.reshape(_B * _P, _D)
    out, mask_i32 = _sc_masked_copy(xf, segs, mval16)
    return out.reshape(_B, _P, _D), mask_i32.reshape(_B, _P) != 0


# speedup vs baseline: 30.8094x; 30.8094x over previous
import functools

import jax
import jax.numpy as jnp
from jax import lax
from jax.experimental import pallas as pl
from jax.experimental.pallas import tpu as pltpu
from jax.experimental.pallas import tpu_sc as plsc

_MAX_MASK_PCT = 0.15
_NUM_MASKS = 2
_B, _P, _D = 16, 2048, 1024
_NW = 32
_RPW = _B * _P // _NW
_BP = 512


def _segment_bounds(X_len):
    rk = jax.random.key(42)
    ka, kb = jax.random.split(rk)
    valid = X_len
    mml = jnp.floor(_MAX_MASK_PCT * valid.astype(jnp.float32)).astype(jnp.int32)
    vrep = jnp.repeat(valid, _NUM_MASKS)
    mrep = jnp.repeat(mml, _NUM_MASKS)
    n = _B * _NUM_MASKS
    t = jnp.floor(jax.random.uniform(ka, (n,)) * (mrep + 1).astype(jnp.float32)).astype(jnp.int32)
    max_start = jnp.clip(vrep - t + 1, 1, None)
    t0 = jnp.floor(jax.random.uniform(kb, (n,)) * max_start.astype(jnp.float32)).astype(jnp.int32)
    t1 = t0 + t
    return jnp.stack(
        [t0.reshape(_B, _NUM_MASKS), t1.reshape(_B, _NUM_MASKS)], axis=-1
    ).reshape(_B, 4)


_mesh = plsc.VectorSubcoreMesh(core_axis_name="c", subcore_axis_name="s")


@functools.partial(
    pl.kernel,
    mesh=_mesh,
    out_type=jax.ShapeDtypeStruct((_B * _P,), jnp.int32),
    scratch_types=[
        pltpu.VMEM((_RPW,), jnp.int32),
        pltpu.VMEM((16,), jnp.int32),
    ],
)
def _sc_mask_build(segs_hbm, mask_hbm, maskbuf, segs_v):
    wid = lax.axis_index("s") * 2 + lax.axis_index("c")
    base = wid * _RPW
    p0 = (wid % 2) * _RPW

    pltpu.sync_copy(segs_hbm.at[wid], segs_v)
    sv = segs_v[:]
    s0 = sv[0]
    e0 = sv[1]
    s1 = sv[2]
    e1 = sv[3]

    one16 = jnp.full((16,), 1, jnp.int32)
    zero16 = jnp.zeros((16,), jnp.int32)

    def mrow(i, c):
        p = p0 + i * 16 + lax.iota(jnp.int32, 16)
        m = ((p >= s0) & (p < e0)) | ((p >= s1) & (p < e1))
        maskbuf[pl.ds(i * 16, 16)] = jnp.where(m, one16, zero16)
        return c

    lax.fori_loop(0, _RPW // 16, mrow, 0)
    pltpu.sync_copy(maskbuf, mask_hbm.at[pl.ds(base, _RPW)])


def _tc_body(segs_ref, mval_ref, x_ref, o_ref):
    b = pl.program_id(0)
    j = pl.program_id(1)
    s0 = segs_ref[4 * b]
    e0 = segs_ref[4 * b + 1]
    s1 = segs_ref[4 * b + 2]
    e1 = segs_ref[4 * b + 3]
    p = j * _BP + lax.broadcasted_iota(jnp.int32, (1, _BP, 1), 1)
    m = ((p >= s0) & (p < e0)) | ((p >= s1) & (p < e1))
    o_ref[...] = jnp.where(m, mval_ref[0], x_ref[...])


_tc_masked_copy = pl.pallas_call(
    _tc_body,
    grid=(_B, _P // _BP),
    in_specs=[
        pl.BlockSpec(memory_space=pltpu.SMEM),
        pl.BlockSpec(memory_space=pltpu.SMEM),
        pl.BlockSpec((1, _BP, _D), lambda b, j: (b, j, 0)),
    ],
    out_specs=pl.BlockSpec((1, _BP, _D), lambda b, j: (b, j, 0)),
    out_shape=jax.ShapeDtypeStruct((_B, _P, _D), jnp.float32),
)


def kernel(X, X_len, mask_value):
    segs = _segment_bounds(X_len)
    segs_w = jnp.repeat(jnp.pad(segs, ((0, 0), (0, 12))), _NW // _B, axis=0)
    mask_i32 = _sc_mask_build(segs_w)
    out = _tc_masked_copy(segs.reshape(_B * 4), mask_value, X)
    return out, mask_i32.reshape(_B, _P) != 0
